# baseline (speedup 1.0000x reference)
import functools

import jax
import jax.numpy as jnp
from jax.experimental import pallas as pl
from jax.experimental.pallas import tpu as pltpu
from jax.experimental.pallas import tpu_sc as plsc

K_NN = 4
BT = 1024


LANES = 128


def srcnorm_body(src_ref, srcn_ref):
    s = src_ref[...]
    n = jnp.sqrt(jnp.sum(s * s, axis=1, keepdims=True)) + 1e-8
    srcn_ref[...] = (s / n).astype(jnp.bfloat16)


def stream_body(srcn_ref, tgt_ref, pv_ref, pi_ref, *, t_total):
    bt = pl.program_id(0)
    q, _ = srcn_ref.shape
    btn = tgt_ref.shape[0]
    groups = btn // LANES

    @pl.when(bt == 0)
    def _init():
        pv_ref[...] = jnp.full(pv_ref.shape, -jnp.inf, jnp.float32)
        pi_ref[...] = jnp.zeros(pi_ref.shape, jnp.int32)

    tb = tgt_ref[...]
    tn = jnp.sqrt(jnp.sum(tb * tb, axis=1, keepdims=True)) + 1e-8
    tbn = (tb / tn).astype(jnp.bfloat16)
    sim = jax.lax.dot_general(
        srcn_ref[...], tbn,
        dimension_numbers=(((1,), (1,)), ((), ())),
        preferred_element_type=jnp.float32,
    )

    a = [pv_ref[:, s * LANES:(s + 1) * LANES] for s in range(K_NN)]
    ix = [pi_ref[:, s * LANES:(s + 1) * LANES] for s in range(K_NN)]
    lane = jax.lax.broadcasted_iota(jnp.int32, (q, LANES), 1)
    for g in range(groups):
        base = bt * btn + g * LANES
        x = sim[:, g * LANES:(g + 1) * LANES]
        x = jnp.where(lane < t_total - base, x, -jnp.inf)
        ordinal = bt * groups + g
        c0 = x > a[0]
        c1 = x > a[1]
        c2 = x > a[2]
        c3 = x > a[3]
        a, ix = (
            [
                jnp.where(c0, x, a[0]),
                jnp.where(c0, a[0], jnp.where(c1, x, a[1])),
                jnp.where(c1, a[1], jnp.where(c2, x, a[2])),
                jnp.where(c2, a[2], jnp.where(c3, x, a[3])),
            ],
            [
                jnp.where(c0, ordinal, ix[0]),
                jnp.where(c0, ix[0], jnp.where(c1, ordinal, ix[1])),
                jnp.where(c1, ix[1], jnp.where(c2, ordinal, ix[2])),
                jnp.where(c2, ix[2], jnp.where(c3, ordinal, ix[3])),
            ],
        )
    for s in range(K_NN):
        pv_ref[:, s * LANES:(s + 1) * LANES] = a[s]
        pi_ref[:, s * LANES:(s + 1) * LANES] = ix[s]


def extract_body(pv_ref, pi_ref, idx_out_ref):
    q, width = pv_ref.shape
    big = jnp.int32(2**30)
    vals = pv_ref[...]
    col = jax.lax.broadcasted_iota(jnp.int32, (q, width), 1)
    idxs = pi_ref[...] * LANES + (col % LANES)
    ri = []
    for _ in range(K_NN):
        m = jnp.max(vals, axis=1, keepdims=True)
        ti = jnp.min(jnp.where(vals == m, idxs, big), axis=1, keepdims=True)
        ri.append(ti)
        vals = jnp.where((vals == m) & (idxs == ti), -jnp.inf, vals)
    ri = jnp.concatenate(ri, axis=1)
    chunks = idx_out_ref.shape[1] // K_NN
    rep = jnp.concatenate([ri] * chunks, axis=1)
    c = jax.lax.broadcasted_iota(jnp.int32, rep.shape, 1) // K_NN
    idx_out_ref[...] = rep * chunks + c


def topk_indices(source_feats, target_feats, interpret=False):
    q, d = source_feats.shape
    t = target_feats.shape[0]
    chunks = d // 128
    nblk = pl.cdiv(t, BT)
    srcn = pl.pallas_call(
        srcnorm_body,
        out_shape=jax.ShapeDtypeStruct((q, d), jnp.bfloat16),
        interpret=interpret,
    )(source_feats)
    pv, pi = pl.pallas_call(
        functools.partial(stream_body, t_total=t),
        grid=(nblk,),
        in_specs=[
            pl.BlockSpec((q, d), lambda i: (0, 0)),
            pl.BlockSpec((BT, d), lambda i: (i, 0)),
        ],
        out_specs=[
            pl.BlockSpec((q, K_NN * LANES), lambda i: (0, 0)),
            pl.BlockSpec((q, K_NN * LANES), lambda i: (0, 0)),
        ],
        out_shape=[
            jax.ShapeDtypeStruct((q, K_NN * LANES), jnp.float32),
            jax.ShapeDtypeStruct((q, K_NN * LANES), jnp.int32),
        ],
        compiler_params=pltpu.CompilerParams(
            dimension_semantics=("arbitrary",),
        ),
        interpret=interpret,
    )(srcn, target_feats)
    return pl.pallas_call(
        extract_body,
        out_shape=jax.ShapeDtypeStruct((q, K_NN * chunks), jnp.int32),
        interpret=interpret,
    )(pv, pi)


def gather_mean(target_feats, idx):
    q = idx.shape[0]
    t, d = target_feats.shape
    chunks = d // 128
    w = 128
    rows_out = w // K_NN
    tgt_view = target_feats.reshape(t * chunks, 128)
    idx_flat = idx.reshape(1, q * K_NN * chunks)
    mesh = plsc.VectorSubcoreMesh(core_axis_name="core", subcore_axis_name="subcore")

    @pl.kernel(
        out_type=jax.ShapeDtypeStruct((q * chunks, 128), jnp.float32),
        mesh=mesh,
        scratch_types=[pltpu.VMEM((w, 128), jnp.float32)],
    )
    def sc_kernel(tgt_hbm, idx_hbm, out_hbm, g_vmem):
        def body(i_vmem, o_vmem):
            pltpu.sync_copy(tgt_hbm.at[i_vmem.at[0]], g_vmem)

            @pl.loop(0, rows_out)
            def _row(r):
                @pl.loop(0, 128, step=16)
                def _col(c):
                    acc = (
                        g_vmem[K_NN * r, pl.ds(c, 16)]
                        + g_vmem[K_NN * r + 1, pl.ds(c, 16)]
                        + g_vmem[K_NN * r + 2, pl.ds(c, 16)]
                        + g_vmem[K_NN * r + 3, pl.ds(c, 16)]
                    )
                    o_vmem[r, pl.ds(c, 16)] = acc * 0.25

        pltpu.emit_pipeline(
            body,
            grid=(q * K_NN * chunks // w,),
            in_specs=[pl.BlockSpec((1, w), lambda i: (0, i))],
            out_specs=[pl.BlockSpec((rows_out, 128), lambda i: (i, 0))],
            core_axis_name=("core", "subcore"),
            dimension_semantics=(pltpu.PARALLEL,),
        )(idx_hbm, out_hbm)

    return sc_kernel(tgt_view, idx_flat).reshape(q, d)


def kernel(source_feats, target_feats):
    idx = topk_indices(source_feats, target_feats)
    return gather_mean(target_feats, idx)


# speedup vs baseline: 1.0080x; 1.0080x over previous
import functools

import jax
import jax.numpy as jnp
from jax.experimental import pallas as pl
from jax.experimental.pallas import tpu as pltpu
from jax.experimental.pallas import tpu_sc as plsc

K_NN = 4
BT = 1024


LANES = 128


def srcnorm_body(src_ref, srcn_ref):
    s = src_ref[...]
    n = jnp.sqrt(jnp.sum(s * s, axis=1, keepdims=True)) + 1e-8
    srcn_ref[...] = (s / n).astype(jnp.bfloat16)


def stream_body(srcn_ref, tgt_ref, idx_out_ref, pv_ref, pi_ref, *, t_total):
    bt = pl.program_id(0)
    q, _ = srcn_ref.shape
    btn = tgt_ref.shape[0]
    groups = btn // LANES

    @pl.when(bt == 0)
    def _init():
        pv_ref[...] = jnp.full(pv_ref.shape, -jnp.inf, jnp.float32)
        pi_ref[...] = jnp.zeros(pi_ref.shape, jnp.int32)

    tb = tgt_ref[...]
    tn = jnp.sqrt(jnp.sum(tb * tb, axis=1, keepdims=True)) + 1e-8
    tbn = (tb / tn).astype(jnp.bfloat16)
    sim = jax.lax.dot_general(
        srcn_ref[...], tbn,
        dimension_numbers=(((1,), (1,)), ((), ())),
        preferred_element_type=jnp.float32,
    )

    a = [pv_ref[:, s * LANES:(s + 1) * LANES] for s in range(K_NN)]
    ix = [pi_ref[:, s * LANES:(s + 1) * LANES] for s in range(K_NN)]
    lane = jax.lax.broadcasted_iota(jnp.int32, (q, LANES), 1)
    for g in range(groups):
        base = bt * btn + g * LANES
        x = sim[:, g * LANES:(g + 1) * LANES]
        x = jnp.where(lane < t_total - base, x, -jnp.inf)
        ordinal = bt * groups + g
        c0 = x > a[0]
        c1 = x > a[1]
        c2 = x > a[2]
        c3 = x > a[3]
        a, ix = (
            [
                jnp.where(c0, x, a[0]),
                jnp.where(c0, a[0], jnp.where(c1, x, a[1])),
                jnp.where(c1, a[1], jnp.where(c2, x, a[2])),
                jnp.where(c2, a[2], jnp.where(c3, x, a[3])),
            ],
            [
                jnp.where(c0, ordinal, ix[0]),
                jnp.where(c0, ix[0], jnp.where(c1, ordinal, ix[1])),
                jnp.where(c1, ix[1], jnp.where(c2, ordinal, ix[2])),
                jnp.where(c2, ix[2], jnp.where(c3, ordinal, ix[3])),
            ],
        )
    for s in range(K_NN):
        pv_ref[:, s * LANES:(s + 1) * LANES] = a[s]
        pi_ref[:, s * LANES:(s + 1) * LANES] = ix[s]

    @pl.when(bt == pl.num_programs(0) - 1)
    def _emit():
        big = jnp.int32(2**30)
        vals = jnp.concatenate(a, axis=1)
        col = jax.lax.broadcasted_iota(jnp.int32, vals.shape, 1)
        idxs = jnp.concatenate(ix, axis=1) * LANES + (col % LANES)
        ri = []
        for _ in range(K_NN):
            m = jnp.max(vals, axis=1, keepdims=True)
            ti = jnp.min(jnp.where(vals == m, idxs, big), axis=1, keepdims=True)
            ri.append(ti)
            vals = jnp.where((vals == m) & (idxs == ti), -jnp.inf, vals)
        ri = jnp.concatenate(ri, axis=1)
        chunks = idx_out_ref.shape[1] // K_NN
        rep = jnp.concatenate([ri] * chunks, axis=1)
        c = jax.lax.broadcasted_iota(jnp.int32, rep.shape, 1) // K_NN
        idx_out_ref[...] = rep * chunks + c


def topk_indices(source_feats, target_feats, interpret=False):
    q, d = source_feats.shape
    t = target_feats.shape[0]
    chunks = d // 128
    nblk = pl.cdiv(t, BT)
    srcn = pl.pallas_call(
        srcnorm_body,
        out_shape=jax.ShapeDtypeStruct((q, d), jnp.bfloat16),
        interpret=interpret,
    )(source_feats)
    return pl.pallas_call(
        functools.partial(stream_body, t_total=t),
        grid=(nblk,),
        in_specs=[
            pl.BlockSpec((q, d), lambda i: (0, 0)),
            pl.BlockSpec((BT, d), lambda i: (i, 0)),
        ],
        out_specs=pl.BlockSpec((q, K_NN * chunks), lambda i: (0, 0)),
        out_shape=jax.ShapeDtypeStruct((q, K_NN * chunks), jnp.int32),
        scratch_shapes=[
            pltpu.VMEM((q, K_NN * LANES), jnp.float32),
            pltpu.VMEM((q, K_NN * LANES), jnp.int32),
        ],
        compiler_params=pltpu.CompilerParams(
            dimension_semantics=("arbitrary",),
        ),
        interpret=interpret,
    )(srcn, target_feats)


def gather_mean(target_feats, idx):
    q = idx.shape[0]
    t, d = target_feats.shape
    chunks = d // 128
    w = 128
    rows_out = w // K_NN
    tgt_view = target_feats.reshape(t * chunks, 128)
    idx_flat = idx.reshape(1, q * K_NN * chunks)
    mesh = plsc.VectorSubcoreMesh(core_axis_name="core", subcore_axis_name="subcore")

    @pl.kernel(
        out_type=jax.ShapeDtypeStruct((q * chunks, 128), jnp.float32),
        mesh=mesh,
        scratch_types=[pltpu.VMEM((w, 128), jnp.float32)],
    )
    def sc_kernel(tgt_hbm, idx_hbm, out_hbm, g_vmem):
        def body(i_vmem, o_vmem):
            pltpu.sync_copy(tgt_hbm.at[i_vmem.at[0]], g_vmem)

            @pl.loop(0, rows_out)
            def _row(r):
                @pl.loop(0, 128, step=16)
                def _col(c):
                    acc = (
                        g_vmem[K_NN * r, pl.ds(c, 16)]
                        + g_vmem[K_NN * r + 1, pl.ds(c, 16)]
                        + g_vmem[K_NN * r + 2, pl.ds(c, 16)]
                        + g_vmem[K_NN * r + 3, pl.ds(c, 16)]
                    )
                    o_vmem[r, pl.ds(c, 16)] = acc * 0.25

        pltpu.emit_pipeline(
            body,
            grid=(q * K_NN * chunks // w,),
            in_specs=[pl.BlockSpec((1, w), lambda i: (0, i))],
            out_specs=[pl.BlockSpec((rows_out, 128), lambda i: (i, 0))],
            core_axis_name=("core", "subcore"),
            dimension_semantics=(pltpu.PARALLEL,),
        )(idx_hbm, out_hbm)

    return sc_kernel(tgt_view, idx_flat).reshape(q, d)


def kernel(source_feats, target_feats):
    idx = topk_indices(source_feats, target_feats)
    return gather_mean(target_feats, idx)


# speedup vs baseline: 1.2378x; 1.2279x over previous
import functools

import jax
import jax.numpy as jnp
from jax.experimental import pallas as pl
from jax.experimental.pallas import tpu as pltpu
from jax.experimental.pallas import tpu_sc as plsc

K_NN = 4
BT = 512


LANES = 128


def srcnorm_body(src_ref, srcn_ref):
    s = src_ref[...]
    n = jnp.sqrt(jnp.sum(s * s, axis=1, keepdims=True)) + 1e-8
    srcn_ref[...] = (s / n).astype(jnp.bfloat16)


CHUNK = 512


def stream_body(srcn_ref, tgt_ref, idx_out_ref, lin_ref, pv_ref, pi_ref, *, t_total):
    bt = pl.program_id(0)
    q, _ = srcn_ref.shape
    btn = tgt_ref.shape[0]
    groups = btn // LANES

    @pl.when(bt == 0)
    def _init():
        pv_ref[...] = jnp.full(pv_ref.shape, -jnp.inf, jnp.float32)
        pi_ref[...] = jnp.zeros(pi_ref.shape, jnp.int32)

    tb = tgt_ref[...]
    wc = lin_ref.shape[1]
    for ch in range(lin_ref.shape[0] // btn):
        lin_ref[ch * btn:(ch + 1) * btn, :] = tb[:, ch * wc:(ch + 1) * wc]
    tn = jnp.sqrt(jnp.sum(tb * tb, axis=1, keepdims=True)) + 1e-8
    tbn = (tb / tn).astype(jnp.bfloat16)
    sim = jax.lax.dot_general(
        srcn_ref[...], tbn,
        dimension_numbers=(((1,), (1,)), ((), ())),
        preferred_element_type=jnp.float32,
    )

    a = [pv_ref[:, s * LANES:(s + 1) * LANES] for s in range(K_NN)]
    ix = [pi_ref[:, s * LANES:(s + 1) * LANES] for s in range(K_NN)]
    lane = jax.lax.broadcasted_iota(jnp.int32, (q, LANES), 1)
    for g in range(groups):
        base = bt * btn + g * LANES
        x = sim[:, g * LANES:(g + 1) * LANES]
        x = jnp.where(lane < t_total - base, x, -jnp.inf)
        ordinal = bt * groups + g
        c0 = x > a[0]
        c1 = x > a[1]
        c2 = x > a[2]
        c3 = x > a[3]
        a, ix = (
            [
                jnp.where(c0, x, a[0]),
                jnp.where(c0, a[0], jnp.where(c1, x, a[1])),
                jnp.where(c1, a[1], jnp.where(c2, x, a[2])),
                jnp.where(c2, a[2], jnp.where(c3, x, a[3])),
            ],
            [
                jnp.where(c0, ordinal, ix[0]),
                jnp.where(c0, ix[0], jnp.where(c1, ordinal, ix[1])),
                jnp.where(c1, ix[1], jnp.where(c2, ordinal, ix[2])),
                jnp.where(c2, ix[2], jnp.where(c3, ordinal, ix[3])),
            ],
        )
    for s in range(K_NN):
        pv_ref[:, s * LANES:(s + 1) * LANES] = a[s]
        pi_ref[:, s * LANES:(s + 1) * LANES] = ix[s]

    @pl.when(bt == pl.num_programs(0) - 1)
    def _emit():
        big = jnp.int32(2**30)
        vals = jnp.concatenate(a, axis=1)
        col = jax.lax.broadcasted_iota(jnp.int32, vals.shape, 1)
        idxs = jnp.concatenate(ix, axis=1) * LANES + (col % LANES)
        ri = []
        for _ in range(K_NN):
            m = jnp.max(vals, axis=1, keepdims=True)
            ti = jnp.min(jnp.where(vals == m, idxs, big), axis=1, keepdims=True)
            ri.append(ti)
            vals = jnp.where((vals == m) & (idxs == ti), -jnp.inf, vals)
        ri = jnp.concatenate(ri, axis=1)
        nchunks = idx_out_ref.shape[1] // K_NN
        rep = jnp.concatenate([ri] * nchunks, axis=1)
        c = jax.lax.broadcasted_iota(jnp.int32, rep.shape, 1) // K_NN
        idx_out_ref[...] = rep + (rep // btn) * (btn * (nchunks - 1)) + c * btn


def topk_indices(source_feats, target_feats, interpret=False):
    q, d = source_feats.shape
    t = target_feats.shape[0]
    wc = min(CHUNK, d)
    nchunks = d // wc
    nblk = pl.cdiv(t, BT)
    srcn = pl.pallas_call(
        srcnorm_body,
        out_shape=jax.ShapeDtypeStruct((q, d), jnp.bfloat16),
        interpret=interpret,
    )(source_feats)
    return pl.pallas_call(
        functools.partial(stream_body, t_total=t),
        grid=(nblk,),
        in_specs=[
            pl.BlockSpec((q, d), lambda i: (0, 0)),
            pl.BlockSpec((BT, d), lambda i: (i, 0)),
        ],
        out_specs=[
            pl.BlockSpec((q, K_NN * nchunks), lambda i: (0, 0)),
            pl.BlockSpec((BT * nchunks, wc), lambda i: (i, 0)),
        ],
        out_shape=[
            jax.ShapeDtypeStruct((q, K_NN * nchunks), jnp.int32),
            jax.ShapeDtypeStruct((nblk * BT * nchunks, wc), jnp.float32),
        ],
        scratch_shapes=[
            pltpu.VMEM((q, K_NN * LANES), jnp.float32),
            pltpu.VMEM((q, K_NN * LANES), jnp.int32),
        ],
        compiler_params=pltpu.CompilerParams(
            dimension_semantics=("arbitrary",),
        ),
        interpret=interpret,
    )(srcn, target_feats)


def gather_mean(lin, idx, d):
    q = idx.shape[0]
    wc = lin.shape[1]
    nchunks = d // wc
    w = 128
    rows_out = w // K_NN
    idx_flat = idx.reshape(1, q * K_NN * nchunks)
    mesh = plsc.VectorSubcoreMesh(core_axis_name="core", subcore_axis_name="subcore")

    @pl.kernel(
        out_type=jax.ShapeDtypeStruct((q * nchunks, wc), jnp.float32),
        mesh=mesh,
        scratch_types=[pltpu.VMEM((w, wc), jnp.float32)],
    )
    def sc_kernel(lin_hbm, idx_hbm, out_hbm, g_vmem):
        def body(i_vmem, o_vmem):
            pltpu.sync_copy(lin_hbm.at[i_vmem.at[0]], g_vmem)

            @pl.loop(0, rows_out)
            def _row(r):
                @pl.loop(0, wc, step=16)
                def _col(c):
                    acc = (
                        g_vmem[K_NN * r, pl.ds(c, 16)]
                        + g_vmem[K_NN * r + 1, pl.ds(c, 16)]
                        + g_vmem[K_NN * r + 2, pl.ds(c, 16)]
                        + g_vmem[K_NN * r + 3, pl.ds(c, 16)]
                    )
                    o_vmem[r, pl.ds(c, 16)] = acc * 0.25

        pltpu.emit_pipeline(
            body,
            grid=(q * K_NN * nchunks // w,),
            in_specs=[pl.BlockSpec((1, w), lambda i: (0, i))],
            out_specs=[pl.BlockSpec((rows_out, wc), lambda i: (i, 0))],
            core_axis_name=("core", "subcore"),
            dimension_semantics=(pltpu.PARALLEL,),
        )(idx_hbm, out_hbm)

    return sc_kernel(lin, idx_flat).reshape(q, d)


def kernel(source_feats, target_feats):
    idx, lin = topk_indices(source_feats, target_feats)
    return gather_mean(lin, idx, target_feats.shape[1])


# speedup vs baseline: 1.2666x; 1.0233x over previous
import functools

import jax
import jax.numpy as jnp
from jax.experimental import pallas as pl
from jax.experimental.pallas import tpu as pltpu
from jax.experimental.pallas import tpu_sc as plsc

K_NN = 4
BT = 512


LANES = 128


def srcnorm_body(src_ref, srcn_ref):
    s = src_ref[...]
    n = jnp.sqrt(jnp.sum(s * s, axis=1, keepdims=True)) + 1e-8
    srcn_ref[...] = (s / n).astype(jnp.bfloat16)


CHUNK = 512


def stream_body(srcn_ref, tgt_ref, idx_out_ref, lin_ref, pv_ref, pi_ref, *, t_total):
    bt = pl.program_id(0)
    q, _ = srcn_ref.shape
    btn = tgt_ref.shape[0]
    groups = btn // LANES

    @pl.when(bt == 0)
    def _init():
        pv_ref[...] = jnp.full(pv_ref.shape, -jnp.inf, jnp.float32)
        pi_ref[...] = jnp.zeros(pi_ref.shape, jnp.int32)

    tb = tgt_ref[...]
    wc = lin_ref.shape[1]
    for ch in range(lin_ref.shape[0] // btn):
        lin_ref[ch * btn:(ch + 1) * btn, :] = tb[:, ch * wc:(ch + 1) * wc]
    tn = jnp.sqrt(jnp.sum(tb * tb, axis=1, keepdims=True)) + 1e-8
    row = jax.lax.broadcasted_iota(jnp.int32, (btn, 1), 0)
    tn = jnp.where(row < t_total - bt * btn, tn, jnp.float32(jnp.nan))
    tbn = (tb / tn).astype(jnp.bfloat16)
    sim = jax.lax.dot_general(
        srcn_ref[...], tbn,
        dimension_numbers=(((1,), (1,)), ((), ())),
        preferred_element_type=jnp.float32,
    )

    a = [pv_ref[:, s * LANES:(s + 1) * LANES] for s in range(K_NN)]
    ix = [pi_ref[:, s * LANES:(s + 1) * LANES] for s in range(K_NN)]
    for g in range(groups):
        x = sim[:, g * LANES:(g + 1) * LANES]
        ordinal = bt * groups + g
        c0 = x > a[0]
        c1 = x > a[1]
        c2 = x > a[2]
        c3 = x > a[3]
        a, ix = (
            [
                jnp.where(c0, x, a[0]),
                jnp.where(c0, a[0], jnp.where(c1, x, a[1])),
                jnp.where(c1, a[1], jnp.where(c2, x, a[2])),
                jnp.where(c2, a[2], jnp.where(c3, x, a[3])),
            ],
            [
                jnp.where(c0, ordinal, ix[0]),
                jnp.where(c0, ix[0], jnp.where(c1, ordinal, ix[1])),
                jnp.where(c1, ix[1], jnp.where(c2, ordinal, ix[2])),
                jnp.where(c2, ix[2], jnp.where(c3, ordinal, ix[3])),
            ],
        )
    for s in range(K_NN):
        pv_ref[:, s * LANES:(s + 1) * LANES] = a[s]
        pi_ref[:, s * LANES:(s + 1) * LANES] = ix[s]

    @pl.when(bt == pl.num_programs(0) - 1)
    def _emit():
        big = jnp.int32(2**30)
        vals = jnp.concatenate(a, axis=1)
        col = jax.lax.broadcasted_iota(jnp.int32, vals.shape, 1)
        idxs = jnp.concatenate(ix, axis=1) * LANES + (col % LANES)
        ri = []
        for _ in range(K_NN):
            m = jnp.max(vals, axis=1, keepdims=True)
            ti = jnp.min(jnp.where(vals == m, idxs, big), axis=1, keepdims=True)
            ri.append(ti)
            vals = jnp.where((vals == m) & (idxs == ti), -jnp.inf, vals)
        ri = jnp.concatenate(ri, axis=1)
        nchunks = idx_out_ref.shape[1] // K_NN
        rep = jnp.concatenate([ri] * nchunks, axis=1)
        c = jax.lax.broadcasted_iota(jnp.int32, rep.shape, 1) // K_NN
        idx_out_ref[...] = rep + (rep // btn) * (btn * (nchunks - 1)) + c * btn


def topk_indices(source_feats, target_feats, interpret=False):
    q, d = source_feats.shape
    t = target_feats.shape[0]
    wc = min(CHUNK, d)
    nchunks = d // wc
    nblk = pl.cdiv(t, BT)
    srcn = pl.pallas_call(
        srcnorm_body,
        out_shape=jax.ShapeDtypeStruct((q, d), jnp.bfloat16),
        interpret=interpret,
    )(source_feats)
    return pl.pallas_call(
        functools.partial(stream_body, t_total=t),
        grid=(nblk,),
        in_specs=[
            pl.BlockSpec((q, d), lambda i: (0, 0)),
            pl.BlockSpec((BT, d), lambda i: (i, 0)),
        ],
        out_specs=[
            pl.BlockSpec((q, K_NN * nchunks), lambda i: (0, 0)),
            pl.BlockSpec((BT * nchunks, wc), lambda i: (i, 0)),
        ],
        out_shape=[
            jax.ShapeDtypeStruct((q, K_NN * nchunks), jnp.int32),
            jax.ShapeDtypeStruct((nblk * BT * nchunks, wc), jnp.float32),
        ],
        scratch_shapes=[
            pltpu.VMEM((q, K_NN * LANES), jnp.float32),
            pltpu.VMEM((q, K_NN * LANES), jnp.int32),
        ],
        compiler_params=pltpu.CompilerParams(
            dimension_semantics=("arbitrary",),
        ),
        interpret=interpret,
    )(srcn, target_feats)


def gather_mean(lin, idx, d):
    q = idx.shape[0]
    wc = lin.shape[1]
    nchunks = d // wc
    w = 128
    rows_out = w // K_NN
    idx_flat = idx.reshape(1, q * K_NN * nchunks)
    mesh = plsc.VectorSubcoreMesh(core_axis_name="core", subcore_axis_name="subcore")

    @pl.kernel(
        out_type=jax.ShapeDtypeStruct((q * nchunks, wc), jnp.float32),
        mesh=mesh,
        scratch_types=[pltpu.VMEM((w, wc), jnp.float32)],
    )
    def sc_kernel(lin_hbm, idx_hbm, out_hbm, g_vmem):
        def body(i_vmem, o_vmem):
            pltpu.sync_copy(lin_hbm.at[i_vmem.at[0]], g_vmem)

            @pl.loop(0, rows_out)
            def _row(r):
                @pl.loop(0, wc, step=16)
                def _col(c):
                    acc = (
                        g_vmem[K_NN * r, pl.ds(c, 16)]
                        + g_vmem[K_NN * r + 1, pl.ds(c, 16)]
                        + g_vmem[K_NN * r + 2, pl.ds(c, 16)]
                        + g_vmem[K_NN * r + 3, pl.ds(c, 16)]
                    )
                    o_vmem[r, pl.ds(c, 16)] = acc * 0.25

        pltpu.emit_pipeline(
            body,
            grid=(q * K_NN * nchunks // w,),
            in_specs=[pl.BlockSpec((1, w), lambda i: (0, i))],
            out_specs=[pl.BlockSpec((rows_out, wc), lambda i: (i, 0))],
            core_axis_name=("core", "subcore"),
            dimension_semantics=(pltpu.PARALLEL,),
        )(idx_hbm, out_hbm)

    return sc_kernel(lin, idx_flat).reshape(q, d)


def kernel(source_feats, target_feats):
    idx, lin = topk_indices(source_feats, target_feats)
    return gather_mean(lin, idx, target_feats.shape[1])


# speedup vs baseline: 1.2674x; 1.0006x over previous
import functools

import jax
import jax.numpy as jnp
from jax.experimental import pallas as pl
from jax.experimental.pallas import tpu as pltpu
from jax.experimental.pallas import tpu_sc as plsc

K_NN = 4
BT = 512


LANES = 128


def srcnorm_body(src_ref, srcn_ref):
    s = src_ref[...]
    n = jnp.sqrt(jnp.sum(s * s, axis=1, keepdims=True)) + 1e-8
    srcn_ref[...] = (s / n).astype(jnp.bfloat16)


CHUNK = 512


def stream_body(srcn_ref, tgt_ref, idx_out_ref, lin_ref, pv_ref, pi_ref, *, t_total):
    bt = pl.program_id(0)
    q, _ = srcn_ref.shape
    btn = tgt_ref.shape[0]
    groups = btn // LANES

    @pl.when(bt == 0)
    def _init():
        pv_ref[...] = jnp.full(pv_ref.shape, -jnp.inf, jnp.float32)
        pi_ref[...] = jnp.zeros(pi_ref.shape, jnp.int32)

    tb = tgt_ref[...]
    wc = lin_ref.shape[1]
    for ch in range(lin_ref.shape[0] // btn):
        lin_ref[ch * btn:(ch + 1) * btn, :] = tb[:, ch * wc:(ch + 1) * wc]
    tn = jnp.sqrt(jnp.sum(tb * tb, axis=1, keepdims=True)) + 1e-8
    row = jax.lax.broadcasted_iota(jnp.int32, (btn, 1), 0)
    tn = jnp.where(row < t_total - bt * btn, tn, jnp.float32(jnp.nan))
    tbn = (tb / tn).astype(jnp.bfloat16)
    sim = jax.lax.dot_general(
        srcn_ref[...], tbn,
        dimension_numbers=(((1,), (1,)), ((), ())),
        preferred_element_type=jnp.float32,
    )

    a = [pv_ref[:, s * LANES:(s + 1) * LANES] for s in range(K_NN)]
    ix = [pi_ref[:, s * LANES:(s + 1) * LANES] for s in range(K_NN)]
    for g in range(groups):
        x = sim[:, g * LANES:(g + 1) * LANES]
        ordinal = bt * groups + g
        c0 = x > a[0]
        c1 = x > a[1]
        c2 = x > a[2]
        c3 = x > a[3]
        a, ix = (
            [
                jnp.where(c0, x, a[0]),
                jnp.where(c0, a[0], jnp.where(c1, x, a[1])),
                jnp.where(c1, a[1], jnp.where(c2, x, a[2])),
                jnp.where(c2, a[2], jnp.where(c3, x, a[3])),
            ],
            [
                jnp.where(c0, ordinal, ix[0]),
                jnp.where(c0, ix[0], jnp.where(c1, ordinal, ix[1])),
                jnp.where(c1, ix[1], jnp.where(c2, ordinal, ix[2])),
                jnp.where(c2, ix[2], jnp.where(c3, ordinal, ix[3])),
            ],
        )
    for s in range(K_NN):
        pv_ref[:, s * LANES:(s + 1) * LANES] = a[s]
        pi_ref[:, s * LANES:(s + 1) * LANES] = ix[s]

    @pl.when(bt == pl.num_programs(0) - 1)
    def _emit():
        big = jnp.int32(2**30)
        vals = jnp.concatenate(a, axis=1)
        col = jax.lax.broadcasted_iota(jnp.int32, vals.shape, 1)
        idxs = jnp.concatenate(ix, axis=1) * LANES + (col % LANES)
        ri = []
        for _ in range(K_NN):
            m = jnp.max(vals, axis=1, keepdims=True)
            ti = jnp.min(jnp.where(vals == m, idxs, big), axis=1, keepdims=True)
            ri.append(ti)
            vals = jnp.where((vals == m) & (idxs == ti), -jnp.inf, vals)
        ri = jnp.concatenate(ri, axis=1)
        nchunks = idx_out_ref.shape[1] // K_NN
        rep = jnp.concatenate([ri] * nchunks, axis=1)
        c = jax.lax.broadcasted_iota(jnp.int32, rep.shape, 1) // K_NN
        idx_out_ref[...] = rep + (rep // btn) * (btn * (nchunks - 1)) + c * btn


def topk_indices(source_feats, target_feats, interpret=False):
    q, d = source_feats.shape
    t = target_feats.shape[0]
    wc = min(CHUNK, d)
    nchunks = d // wc
    nblk = pl.cdiv(t, BT)
    srcn = pl.pallas_call(
        srcnorm_body,
        out_shape=jax.ShapeDtypeStruct((q, d), jnp.bfloat16),
        interpret=interpret,
    )(source_feats)
    return pl.pallas_call(
        functools.partial(stream_body, t_total=t),
        grid=(nblk,),
        in_specs=[
            pl.BlockSpec((q, d), lambda i: (0, 0)),
            pl.BlockSpec((BT, d), lambda i: (i, 0)),
        ],
        out_specs=[
            pl.BlockSpec((q, K_NN * nchunks), lambda i: (0, 0)),
            pl.BlockSpec((BT * nchunks, wc), lambda i: (i, 0)),
        ],
        out_shape=[
            jax.ShapeDtypeStruct((q, K_NN * nchunks), jnp.int32),
            jax.ShapeDtypeStruct((nblk * BT * nchunks, wc), jnp.float32),
        ],
        scratch_shapes=[
            pltpu.VMEM((q, K_NN * LANES), jnp.float32),
            pltpu.VMEM((q, K_NN * LANES), jnp.int32),
        ],
        compiler_params=pltpu.CompilerParams(
            dimension_semantics=("arbitrary",),
        ),
        interpret=interpret,
    )(srcn, target_feats)


def gather_mean(lin, idx, d):
    q = idx.shape[0]
    wc = lin.shape[1]
    nchunks = d // wc
    w = 128
    rows_out = w // K_NN
    idx_flat = idx.reshape(1, q * K_NN * nchunks)
    mesh = plsc.VectorSubcoreMesh(core_axis_name="core", subcore_axis_name="subcore")

    @pl.kernel(
        out_type=jax.ShapeDtypeStruct((q * nchunks, wc), jnp.float32),
        mesh=mesh,
        scratch_types=[pltpu.VMEM((w, wc), jnp.float32)],
    )
    def sc_kernel(lin_hbm, idx_hbm, out_hbm, g_vmem):
        def body(i_vmem, o_vmem):
            pltpu.sync_copy(lin_hbm.at[i_vmem.at[0]], g_vmem)

            @pl.loop(0, rows_out)
            def _row(r):
                for c in range(0, wc, 16):
                    acc = (
                        g_vmem[K_NN * r, pl.ds(c, 16)]
                        + g_vmem[K_NN * r + 1, pl.ds(c, 16)]
                        + g_vmem[K_NN * r + 2, pl.ds(c, 16)]
                        + g_vmem[K_NN * r + 3, pl.ds(c, 16)]
                    )
                    o_vmem[r, pl.ds(c, 16)] = acc * 0.25

        pltpu.emit_pipeline(
            body,
            grid=(q * K_NN * nchunks // w,),
            in_specs=[pl.BlockSpec((1, w), lambda i: (0, i))],
            out_specs=[pl.BlockSpec((rows_out, wc), lambda i: (i, 0))],
            core_axis_name=("core", "subcore"),
            dimension_semantics=(pltpu.PARALLEL,),
        )(idx_hbm, out_hbm)

    return sc_kernel(lin, idx_flat).reshape(q, d)


def kernel(source_feats, target_feats):
    idx, lin = topk_indices(source_feats, target_feats)
    return gather_mean(lin, idx, target_feats.shape[1])


# speedup vs baseline: 1.2990x; 1.0250x over previous
import functools

import jax
import jax.numpy as jnp
from jax.experimental import pallas as pl
from jax.experimental.pallas import tpu as pltpu
from jax.experimental.pallas import tpu_sc as plsc

K_NN = 4
BT = 512


LANES = 128


def srcnorm_body(src_ref, srcn_ref):
    s = src_ref[...]
    n = jnp.sqrt(jnp.sum(s * s, axis=1, keepdims=True)) + 1e-8
    srcn_ref[...] = (s / n).astype(jnp.bfloat16)


CHUNK = 512


def stream_body(srcn_ref, tgt_ref, idx_out_ref, lin_ref, pv_ref, pi_ref, *, t_total,
                emit_lin=True):
    bt = pl.program_id(0)
    q, _ = srcn_ref.shape
    btn = tgt_ref.shape[0]
    groups = btn // LANES

    @pl.when(bt == 0)
    def _init():
        pv_ref[...] = jnp.full(pv_ref.shape, -jnp.inf, jnp.float32)
        pi_ref[...] = jnp.zeros(pi_ref.shape, jnp.int32)

    tb = tgt_ref[...]
    if emit_lin:
        wc = lin_ref.shape[1]
        for ch in range(lin_ref.shape[0] // btn):
            lin_ref[ch * btn:(ch + 1) * btn, :] = tb[:, ch * wc:(ch + 1) * wc]
    tn = jnp.sqrt(jnp.sum(tb * tb, axis=1, keepdims=True)) + 1e-8
    row = jax.lax.broadcasted_iota(jnp.int32, (btn, 1), 0)
    tn = jnp.where(row < t_total - bt * btn, tn, jnp.float32(jnp.nan))
    tbn = (tb / tn).astype(jnp.bfloat16)
    sim = jax.lax.dot_general(
        srcn_ref[...], tbn,
        dimension_numbers=(((1,), (1,)), ((), ())),
        preferred_element_type=jnp.float32,
    )

    a = [pv_ref[:, s * LANES:(s + 1) * LANES] for s in range(K_NN)]
    ix = [pi_ref[:, s * LANES:(s + 1) * LANES] for s in range(K_NN)]
    for g in range(groups):
        x = sim[:, g * LANES:(g + 1) * LANES]
        ordinal = bt * groups + g
        c0 = x > a[0]
        c1 = x > a[1]
        c2 = x > a[2]
        c3 = x > a[3]
        a, ix = (
            [
                jnp.where(c0, x, a[0]),
                jnp.where(c0, a[0], jnp.where(c1, x, a[1])),
                jnp.where(c1, a[1], jnp.where(c2, x, a[2])),
                jnp.where(c2, a[2], jnp.where(c3, x, a[3])),
            ],
            [
                jnp.where(c0, ordinal, ix[0]),
                jnp.where(c0, ix[0], jnp.where(c1, ordinal, ix[1])),
                jnp.where(c1, ix[1], jnp.where(c2, ordinal, ix[2])),
                jnp.where(c2, ix[2], jnp.where(c3, ordinal, ix[3])),
            ],
        )
    for s in range(K_NN):
        pv_ref[:, s * LANES:(s + 1) * LANES] = a[s]
        pi_ref[:, s * LANES:(s + 1) * LANES] = ix[s]

    @pl.when(bt == pl.num_programs(0) - 1)
    def _emit():
        big = jnp.int32(2**30)
        vals = jnp.concatenate(a, axis=1)
        col = jax.lax.broadcasted_iota(jnp.int32, vals.shape, 1)
        idxs = jnp.concatenate(ix, axis=1) * LANES + (col % LANES)
        ri = []
        for _ in range(K_NN):
            m = jnp.max(vals, axis=1, keepdims=True)
            ti = jnp.min(jnp.where(vals == m, idxs, big), axis=1, keepdims=True)
            ri.append(ti)
            vals = jnp.where((vals == m) & (idxs == ti), -jnp.inf, vals)
        ri = jnp.concatenate(ri, axis=1)
        nchunks = idx_out_ref.shape[1] // K_NN
        rep = jnp.concatenate([ri] * nchunks, axis=1)
        c = jax.lax.broadcasted_iota(jnp.int32, rep.shape, 1) // K_NN
        idx_out_ref[...] = rep + (rep // btn) * (btn * (nchunks - 1)) + c * btn


def stream_body_nolin(srcn_ref, tgt_ref, idx_out_ref, pv_ref, pi_ref, *, t_total):
    stream_body(srcn_ref, tgt_ref, idx_out_ref, None, pv_ref, pi_ref,
                t_total=t_total, emit_lin=False)


def topk_indices(source_feats, target_feats, interpret=False):
    q, d = source_feats.shape
    t = target_feats.shape[0]
    wc = min(CHUNK, d)
    nchunks = d // wc
    nblk = pl.cdiv(t, BT)
    qh = q // 2
    srcn = pl.pallas_call(
        srcnorm_body,
        out_shape=jax.ShapeDtypeStruct((q, d), jnp.bfloat16),
        interpret=interpret,
    )(source_feats)

    def half_call(h, with_lin):
        out_specs = [pl.BlockSpec((qh, K_NN * nchunks), lambda i: (0, 0))]
        out_shape = [jax.ShapeDtypeStruct((qh, K_NN * nchunks), jnp.int32)]
        if with_lin:
            out_specs.append(pl.BlockSpec((BT * nchunks, wc), lambda i: (i, 0)))
            out_shape.append(
                jax.ShapeDtypeStruct((nblk * BT * nchunks, wc), jnp.float32))
            body = functools.partial(stream_body, t_total=t)
        else:
            body = functools.partial(stream_body_nolin, t_total=t)
        return pl.pallas_call(
            body,
            grid=(nblk,),
            in_specs=[
                pl.BlockSpec((qh, d), lambda i, h=h: (h, 0)),
                pl.BlockSpec((BT, d), lambda i: (i, 0)),
            ],
            out_specs=out_specs,
            out_shape=out_shape,
            scratch_shapes=[
                pltpu.VMEM((qh, K_NN * LANES), jnp.float32),
                pltpu.VMEM((qh, K_NN * LANES), jnp.int32),
            ],
            compiler_params=pltpu.CompilerParams(
                dimension_semantics=("arbitrary",),
            ),
            interpret=interpret,
        )(srcn, target_feats)

    idx1, lin = half_call(0, True)
    (idx2,) = half_call(1, False)
    return (idx1, idx2), lin


def gather_mean(lin, idx, d):
    q = idx.shape[0]
    wc = lin.shape[1]
    nchunks = d // wc
    w = 128
    rows_out = w // K_NN
    idx_flat = idx.reshape(1, q * K_NN * nchunks)
    mesh = plsc.VectorSubcoreMesh(core_axis_name="core", subcore_axis_name="subcore")

    @pl.kernel(
        out_type=jax.ShapeDtypeStruct((q * nchunks, wc), jnp.float32),
        mesh=mesh,
        scratch_types=[pltpu.VMEM((w, wc), jnp.float32)],
    )
    def sc_kernel(lin_hbm, idx_hbm, out_hbm, g_vmem):
        def body(i_vmem, o_vmem):
            pltpu.sync_copy(lin_hbm.at[i_vmem.at[0]], g_vmem)

            @pl.loop(0, rows_out)
            def _row(r):
                for c in range(0, wc, 16):
                    acc = (
                        g_vmem[K_NN * r, pl.ds(c, 16)]
                        + g_vmem[K_NN * r + 1, pl.ds(c, 16)]
                        + g_vmem[K_NN * r + 2, pl.ds(c, 16)]
                        + g_vmem[K_NN * r + 3, pl.ds(c, 16)]
                    )
                    o_vmem[r, pl.ds(c, 16)] = acc * 0.25

        pltpu.emit_pipeline(
            body,
            grid=(q * K_NN * nchunks // w,),
            in_specs=[pl.BlockSpec((1, w), lambda i: (0, i))],
            out_specs=[pl.BlockSpec((rows_out, wc), lambda i: (i, 0))],
            core_axis_name=("core", "subcore"),
            dimension_semantics=(pltpu.PARALLEL,),
        )(idx_hbm, out_hbm)

    return sc_kernel(lin, idx_flat).reshape(q, d)


def kernel(source_feats, target_feats):
    (idx1, idx2), lin = topk_indices(source_feats, target_feats)
    d = target_feats.shape[1]
    out1 = gather_mean(lin, idx1, d)
    out2 = gather_mean(lin, idx2, d)
    return jnp.concatenate([out1, out2], axis=0)


# speedup vs baseline: 1.5091x; 1.1617x over previous
import functools

import jax
import jax.numpy as jnp
from jax.experimental import pallas as pl
from jax.experimental.pallas import tpu as pltpu
from jax.experimental.pallas import tpu_sc as plsc

K_NN = 4
BT = 1024


LANES = 128


def srcnorm_body(src_ref, srcn_ref):
    s = src_ref[...]
    n = jnp.sqrt(jnp.sum(s * s, axis=1, keepdims=True)) + 1e-8
    srcn_ref[...] = (s / n).astype(jnp.bfloat16)


CHUNK = 512


def stream_body(srcn_ref, tgt_ref, idx_out_ref, lin_ref, pv_ref, pi_ref, *, t_total,
                emit_lin=True):
    bt = pl.program_id(0)
    q, _ = srcn_ref.shape
    btn = tgt_ref.shape[0]
    groups = btn // LANES

    @pl.when(bt == 0)
    def _init():
        pv_ref[...] = jnp.full(pv_ref.shape, -jnp.inf, jnp.float32)
        pi_ref[...] = jnp.zeros(pi_ref.shape, jnp.int32)

    tb = tgt_ref[...]
    if emit_lin:
        wc = lin_ref.shape[1]
        for ch in range(lin_ref.shape[0] // btn):
            lin_ref[ch * btn:(ch + 1) * btn, :] = tb[:, ch * wc:(ch + 1) * wc]
    tn = jnp.sqrt(jnp.sum(tb * tb, axis=1, keepdims=True)) + 1e-8
    row = jax.lax.broadcasted_iota(jnp.int32, (btn, 1), 0)
    tn = jnp.where(row < t_total - bt * btn, tn, jnp.float32(jnp.nan))
    tbn = (tb / tn).astype(jnp.bfloat16)
    sim = jax.lax.dot_general(
        srcn_ref[...], tbn,
        dimension_numbers=(((1,), (1,)), ((), ())),
        preferred_element_type=jnp.float32,
    )

    a = [pv_ref[:, s * LANES:(s + 1) * LANES] for s in range(K_NN)]
    ix = [pi_ref[:, s * LANES:(s + 1) * LANES] for s in range(K_NN)]
    for g in range(groups):
        x = sim[:, g * LANES:(g + 1) * LANES]
        ordinal = bt * groups + g
        c0 = x > a[0]
        c1 = x > a[1]
        c2 = x > a[2]
        c3 = x > a[3]
        a, ix = (
            [
                jnp.where(c0, x, a[0]),
                jnp.where(c0, a[0], jnp.where(c1, x, a[1])),
                jnp.where(c1, a[1], jnp.where(c2, x, a[2])),
                jnp.where(c2, a[2], jnp.where(c3, x, a[3])),
            ],
            [
                jnp.where(c0, ordinal, ix[0]),
                jnp.where(c0, ix[0], jnp.where(c1, ordinal, ix[1])),
                jnp.where(c1, ix[1], jnp.where(c2, ordinal, ix[2])),
                jnp.where(c2, ix[2], jnp.where(c3, ordinal, ix[3])),
            ],
        )
    for s in range(K_NN):
        pv_ref[:, s * LANES:(s + 1) * LANES] = a[s]
        pi_ref[:, s * LANES:(s + 1) * LANES] = ix[s]

    @pl.when(bt == pl.num_programs(0) - 1)
    def _emit():
        big = jnp.int32(2**30)
        vals = jnp.concatenate(a, axis=1)
        col = jax.lax.broadcasted_iota(jnp.int32, vals.shape, 1)
        idxs = jnp.concatenate(ix, axis=1) * LANES + (col % LANES)
        ri = []
        for _ in range(K_NN):
            m = jnp.max(vals, axis=1, keepdims=True)
            ti = jnp.min(jnp.where(vals == m, idxs, big), axis=1, keepdims=True)
            ri.append(ti)
            vals = jnp.where((vals == m) & (idxs == ti), -jnp.inf, vals)
        ri = jnp.concatenate(ri, axis=1)
        nchunks = idx_out_ref.shape[1] // K_NN
        rep = jnp.concatenate([ri] * nchunks, axis=1)
        c = jax.lax.broadcasted_iota(jnp.int32, rep.shape, 1) // K_NN
        idx_out_ref[...] = rep + (rep // btn) * (btn * (nchunks - 1)) + c * btn


def stream_body_nolin(srcn_ref, tgt_ref, idx_out_ref, pv_ref, pi_ref, *, t_total):
    stream_body(srcn_ref, tgt_ref, idx_out_ref, None, pv_ref, pi_ref,
                t_total=t_total, emit_lin=False)


def topk_indices(source_feats, target_feats, interpret=False):
    q, d = source_feats.shape
    t = target_feats.shape[0]
    wc = min(CHUNK, d)
    nchunks = d // wc
    nblk = pl.cdiv(t, BT)
    qh = q // 2
    srcn = pl.pallas_call(
        srcnorm_body,
        out_shape=jax.ShapeDtypeStruct((q, d), jnp.bfloat16),
        interpret=interpret,
    )(source_feats)

    def half_call(h, with_lin):
        out_specs = [pl.BlockSpec((qh, K_NN * nchunks), lambda i: (0, 0))]
        out_shape = [jax.ShapeDtypeStruct((qh, K_NN * nchunks), jnp.int32)]
        if with_lin:
            out_specs.append(pl.BlockSpec((BT * nchunks, wc), lambda i: (i, 0)))
            out_shape.append(
                jax.ShapeDtypeStruct((nblk * BT * nchunks, wc), jnp.float32))
            body = functools.partial(stream_body, t_total=t)
        else:
            body = functools.partial(stream_body_nolin, t_total=t)
        return pl.pallas_call(
            body,
            grid=(nblk,),
            in_specs=[
                pl.BlockSpec((qh, d), lambda i, h=h: (h, 0)),
                pl.BlockSpec((BT, d), lambda i: (i, 0)),
            ],
            out_specs=out_specs,
            out_shape=out_shape,
            scratch_shapes=[
                pltpu.VMEM((qh, K_NN * LANES), jnp.float32),
                pltpu.VMEM((qh, K_NN * LANES), jnp.int32),
            ],
            compiler_params=pltpu.CompilerParams(
                dimension_semantics=("arbitrary",),
            ),
            interpret=interpret,
        )(srcn, target_feats)

    idx1, lin = half_call(0, True)
    (idx2,) = half_call(1, False)
    return (idx1, idx2), lin


def gather_mean(lin, idx, d):
    q = idx.shape[0]
    wc = lin.shape[1]
    nchunks = d // wc
    w = 128
    rows_out = w // K_NN
    idx_flat = idx.reshape(1, q * K_NN * nchunks)
    mesh = plsc.VectorSubcoreMesh(core_axis_name="core", subcore_axis_name="subcore")

    @pl.kernel(
        out_type=jax.ShapeDtypeStruct((q * nchunks, wc), jnp.float32),
        mesh=mesh,
        scratch_types=[pltpu.VMEM((w, wc), jnp.float32)],
    )
    def sc_kernel(lin_hbm, idx_hbm, out_hbm, g_vmem):
        def body(i_vmem, o_vmem):
            pltpu.sync_copy(lin_hbm.at[i_vmem.at[0]], g_vmem)

            @pl.loop(0, rows_out)
            def _row(r):
                for c in range(0, wc, 16):
                    acc = (
                        g_vmem[K_NN * r, pl.ds(c, 16)]
                        + g_vmem[K_NN * r + 1, pl.ds(c, 16)]
                        + g_vmem[K_NN * r + 2, pl.ds(c, 16)]
                        + g_vmem[K_NN * r + 3, pl.ds(c, 16)]
                    )
                    o_vmem[r, pl.ds(c, 16)] = acc * 0.25

        pltpu.emit_pipeline(
            body,
            grid=(q * K_NN * nchunks // w,),
            in_specs=[pl.BlockSpec((1, w), lambda i: (0, i))],
            out_specs=[pl.BlockSpec((rows_out, wc), lambda i: (i, 0))],
            core_axis_name=("core", "subcore"),
            dimension_semantics=(pltpu.PARALLEL,),
        )(idx_hbm, out_hbm)

    return sc_kernel(lin, idx_flat).reshape(q, d)


def kernel(source_feats, target_feats):
    (idx1, idx2), lin = topk_indices(source_feats, target_feats)
    d = target_feats.shape[1]
    out1 = gather_mean(lin, idx1, d)
    out2 = gather_mean(lin, idx2, d)
    return jnp.concatenate([out1, out2], axis=0)
